# baseline (speedup 1.0000x reference)
import functools

import jax
import jax.numpy as jnp
from jax import lax
from jax.experimental import pallas as pl
from jax.experimental.pallas import tpu as pltpu
from jax.experimental.pallas import tpu_sc as plsc

NC = 2
NS = 16
L = 16
NW = NC * NS

_TOFF = (0, 1, 3, 4, 6, 7, 9, 10, 12, 13)
NF = 10
TR = 8
TLC = 128
WT = 2
UW = WT * TLC


def _tree_sum(xs):
    xs = list(xs)
    while len(xs) > 1:
        nxt = [a + b for a, b in zip(xs[::2], xs[1::2])]
        if len(xs) % 2:
            nxt.append(xs[-1])
        xs = nxt
    return xs[0]


def _sc_loss_body(NA, NB, pred, targ, score, mask, stride, out,
                  pbuf, tbuf, sbuf, mbuf, stbuf, ostage, sem0, sem1):
    nrt = NB // TR
    ncu = (NA + UW - 1) // UW
    NU = nrt * ncu
    per = NU // NW
    SLOTS = per + 2
    wid = lax.axis_index("s") * NC + lax.axis_index("c")
    u0 = (wid * NU) // NW
    cnt = ((wid + 1) * NU) // NW - u0
    sems = (sem0, sem1)

    def unit_of(k):
        u = u0 + jnp.minimum(k, cnt - 1)
        tc = u // nrt
        tr = u - tc * nrt
        return tr * TR, tc * UW, tc

    def issue(k, b):
        ro, co, _ = unit_of(k)
        sm = sems[b]
        for p in range(NF):
            pltpu.async_copy(pred.at[p, pl.ds(ro, TR), pl.ds(co, UW)],
                             pbuf.at[b, p], sm)
            pltpu.async_copy(targ.at[_TOFF[p], pl.ds(ro, TR), pl.ds(co, UW)],
                             tbuf.at[b, p], sm)
        pltpu.async_copy(score.at[pl.ds(ro, TR), pl.ds(co, UW)], sbuf.at[b], sm)
        pltpu.async_copy(mask.at[pl.ds(ro, TR), pl.ds(co, UW)], mbuf.at[b], sm)
        pltpu.async_copy(stride.at[pl.ds(co, UW)], stbuf.at[b], sm)

    def drain(b):
        sm = sems[b]
        for p in range(NF):
            pltpu.make_async_copy(pred.at[0, pl.ds(0, TR), pl.ds(0, UW)],
                                  pbuf.at[b, p], sm).wait()
            pltpu.make_async_copy(targ.at[0, pl.ds(0, TR), pl.ds(0, UW)],
                                  tbuf.at[b, p], sm).wait()
        pltpu.make_async_copy(score.at[pl.ds(0, TR), pl.ds(0, UW)],
                              sbuf.at[b], sm).wait()
        pltpu.make_async_copy(mask.at[pl.ds(0, TR), pl.ds(0, UW)],
                              mbuf.at[b], sm).wait()
        pltpu.make_async_copy(stride.at[pl.ds(0, UW)], stbuf.at[b], sm).wait()

    def compute(k, b, al, an):
        _, _, tc = unit_of(k)
        vc = jnp.where(k < cnt,
                       jnp.minimum(NA - tc * UW, UW) // L,
                       0)

        def jbody(j, carry):
            al, an = carry
            co = j * L
            si = 1.0 / stbuf[b, pl.ds(co, L)]
            for r in range(TR):
                mk = mbuf[b, r, pl.ds(co, L)]
                w = sbuf[b, r, pl.ds(co, L)] * mk
                an = an + mk
                s = None
                for p in range(0, NF, 2):
                    tp = None
                    for q in (p, p + 1):
                        pp = pbuf[b, q, r, pl.ds(co, L)]
                        tt = tbuf[b, q, r, pl.ds(co, L)]
                        d = jnp.abs(pp - tt * si)
                        m = jnp.minimum(d, 1.0)
                        t = (d - 0.5 * m) * m
                        tp = t if tp is None else tp + t
                    s = tp if s is None else s + tp
                al = al + s * w
            return al, an

        return lax.fori_loop(0, vc, jbody, (al, an))

    issue(jnp.int32(0), 0)

    def pair(kp, carry):
        al, an = carry
        for b in (0, 1):
            k = kp * 2 + b

            @pl.when(k + 1 < SLOTS)
            def _():
                issue(k + 1, 1 - b)

            drain(b)
            al, an = compute(k, b, al, an)
        return al, an

    al = jnp.zeros((L,), jnp.float32)
    an = jnp.zeros((L,), jnp.float32)
    al, an = lax.fori_loop(0, SLOTS // 2, pair, (al, an))

    ostage[pl.ds(0, L)] = al
    ostage[pl.ds(L, L)] = an
    pltpu.sync_copy(ostage, out.at[pl.ds(wid * 2 * L, 2 * L)])


@functools.partial(jax.jit, static_argnums=(5, 6))
def _sc_loss(pred, targ, score, mask, stride, NA, NB):
    mesh = plsc.VectorSubcoreMesh(core_axis_name="c", subcore_axis_name="s",
                                  num_cores=NC, num_subcores=NS)
    body = functools.partial(_sc_loss_body, NA, NB)
    f = pl.kernel(
        body,
        out_type=jax.ShapeDtypeStruct((NW * 2 * L,), jnp.float32),
        mesh=mesh,
        scratch_types=[
            pltpu.VMEM((2, NF, TR, UW), jnp.float32),
            pltpu.VMEM((2, NF, TR, UW), jnp.float32),
            pltpu.VMEM((2, TR, UW), jnp.float32),
            pltpu.VMEM((2, TR, UW), jnp.float32),
            pltpu.VMEM((2, UW), jnp.float32),
            pltpu.VMEM((2 * L,), jnp.float32),
            pltpu.SemaphoreType.DMA,
            pltpu.SemaphoreType.DMA,
        ],
        compiler_params=pltpu.CompilerParams(
            needs_layout_passes=False,
            use_tc_tiling_on_sc=True,
            disable_bounds_checks=True,
        ),
    )
    return f(pred, targ, score, mask, stride)


def kernel(pred_kps, target_kps, stride_tensor, target_scores,
           target_scores_sum, fg_mask):
    bs, na = fg_mask.shape

    pred = pred_kps.transpose(2, 0, 1)
    targ = target_kps.transpose(2, 0, 1)
    score = target_scores.reshape(bs, na)
    mask = fg_mask.astype(jnp.float32)
    stride = stride_tensor.reshape(-1)

    o = _sc_loss(pred, targ, score, mask, stride, na, bs).reshape(NW, 2, L)
    loss_sum = o[:, 0].sum()
    num_pos = o[:, 1].sum()
    denom = num_pos * 10.0
    safe = jnp.where(denom == 0.0, jnp.float32(1.0), denom)
    l = loss_sum / safe
    ts = target_scores_sum.reshape(())
    lpos = jnp.where(ts == 0.0, l, l / ts)
    return jnp.where(num_pos > 0.0, lpos, jnp.float32(0.0))


# speedup vs baseline: 3.6795x; 3.6795x over previous
import functools

import jax
import jax.numpy as jnp
from jax import lax
from jax.experimental import pallas as pl
from jax.experimental.pallas import tpu as pltpu
from jax.experimental.pallas import tpu_sc as plsc

NC = 2
NS = 16
L = 16
NW = NC * NS

_TOFF = (0, 1, 3, 4, 6, 7, 9, 10, 12, 13)
NF = 10
TR = 8
TLC = 128


def _sc_loss_body(NA, NB, pred, targ, score, mask, stride, out,
                  pbuf, tbuf, sbuf, mbuf, stbuf, ostage, sem0, sem1):
    nrt = NB // TR
    nct = (NA + TLC - 1) // TLC
    NU = nrt * nct
    SLOTS = (NU // NW) + 2
    wid = lax.axis_index("s") * NC + lax.axis_index("c")
    u0 = (wid * NU) // NW
    cnt = ((wid + 1) * NU) // NW - u0
    sems = (sem0, sem1)

    def unit_of(k):
        u = u0 + jnp.minimum(k, cnt - 1)
        tc = u // nrt
        tr = u - tc * nrt
        return tr * TR, tc * TLC, tc

    def issue(k, b):
        ro, co, _ = unit_of(k)
        sm = sems[b]
        for p in range(NF):
            pltpu.async_copy(pred.at[p, pl.ds(ro, TR), pl.ds(co, TLC)],
                             pbuf.at[b, p], sm)
            pltpu.async_copy(targ.at[_TOFF[p], pl.ds(ro, TR), pl.ds(co, TLC)],
                             tbuf.at[b, p], sm)
        pltpu.async_copy(score.at[pl.ds(ro, TR), pl.ds(co, TLC)], sbuf.at[b], sm)
        pltpu.async_copy(mask.at[pl.ds(ro, TR), pl.ds(co, TLC)], mbuf.at[b], sm)
        pltpu.async_copy(stride.at[pl.ds(co, TLC)], stbuf.at[b], sm)

    def drain(b):
        sm = sems[b]
        pltpu.make_async_copy(
            pred.at[pl.ds(0, NF), pl.ds(0, TR), pl.ds(0, TLC)],
            pbuf.at[b], sm).wait()
        pltpu.make_async_copy(
            targ.at[pl.ds(0, NF), pl.ds(0, TR), pl.ds(0, TLC)],
            tbuf.at[b], sm).wait()
        pltpu.make_async_copy(score.at[pl.ds(0, TR), pl.ds(0, TLC)],
                              sbuf.at[b], sm).wait()
        pltpu.make_async_copy(mask.at[pl.ds(0, TR), pl.ds(0, TLC)],
                              mbuf.at[b], sm).wait()
        pltpu.make_async_copy(stride.at[pl.ds(0, TLC)], stbuf.at[b], sm).wait()

    def compute(k, b, al, an):
        _, _, tc = unit_of(k)
        vc = jnp.where(k < cnt,
                       jnp.minimum(NA - tc * TLC, TLC) // L,
                       0)

        def jbody(ch, carry):
            al, an = carry
            r = ch // vc
            co = (ch - r * vc) * L
            si = 1.0 / stbuf[b, pl.ds(co, L)]
            mk = mbuf[b, r, pl.ds(co, L)]
            w = sbuf[b, r, pl.ds(co, L)] * mk
            an = an + mk
            s = None
            for p in range(0, NF, 2):
                tp = None
                for q in (p, p + 1):
                    pp = pbuf[b, q, r, pl.ds(co, L)]
                    tt = tbuf[b, q, r, pl.ds(co, L)]
                    d = jnp.abs(pp - tt * si)
                    m = jnp.minimum(d, 1.0)
                    t = (d - 0.5 * m) * m
                    tp = t if tp is None else tp + t
                s = tp if s is None else s + tp
            al = al + s * w
            return al, an

        return lax.fori_loop(0, vc * TR, jbody, (al, an))

    issue(jnp.int32(0), 0)

    def pair(kp, carry):
        al, an = carry
        for b in (0, 1):
            k = kp * 2 + b

            @pl.when(k + 1 < SLOTS)
            def _():
                issue(k + 1, 1 - b)

            drain(b)
            al, an = compute(k, b, al, an)
        return al, an

    al = jnp.zeros((L,), jnp.float32)
    an = jnp.zeros((L,), jnp.float32)
    al, an = lax.fori_loop(0, SLOTS // 2, pair, (al, an))

    ostage[pl.ds(0, L)] = al
    ostage[pl.ds(L, L)] = an
    pltpu.sync_copy(ostage, out.at[pl.ds(wid * 2 * L, 2 * L)])


@functools.partial(jax.jit, static_argnums=(5, 6))
def _sc_loss(pred, targ, score, mask, stride, NA, NB):
    mesh = plsc.VectorSubcoreMesh(core_axis_name="c", subcore_axis_name="s",
                                  num_cores=NC, num_subcores=NS)
    body = functools.partial(_sc_loss_body, NA, NB)
    f = pl.kernel(
        body,
        out_type=jax.ShapeDtypeStruct((NW * 2 * L,), jnp.float32),
        mesh=mesh,
        scratch_types=[
            pltpu.VMEM((2, NF, TR, TLC), jnp.float32),
            pltpu.VMEM((2, NF, TR, TLC), jnp.float32),
            pltpu.VMEM((2, TR, TLC), jnp.float32),
            pltpu.VMEM((2, TR, TLC), jnp.float32),
            pltpu.VMEM((2, TLC), jnp.float32),
            pltpu.VMEM((2 * L,), jnp.float32),
            pltpu.SemaphoreType.DMA,
            pltpu.SemaphoreType.DMA,
        ],
        compiler_params=pltpu.CompilerParams(
            needs_layout_passes=False,
            use_tc_tiling_on_sc=True,
            disable_bounds_checks=True,
        ),
    )
    return f(pred, targ, score, mask, stride)


def kernel(pred_kps, target_kps, stride_tensor, target_scores,
           target_scores_sum, fg_mask):
    bs, na = fg_mask.shape

    pred = pred_kps.transpose(2, 0, 1)
    targ = target_kps.transpose(2, 0, 1)
    score = target_scores.reshape(bs, na)
    mask = fg_mask.astype(jnp.float32)
    stride = stride_tensor.reshape(-1)

    o = _sc_loss(pred, targ, score, mask, stride, na, bs).reshape(NW, 2, L)
    loss_sum = o[:, 0].sum()
    num_pos = o[:, 1].sum()
    denom = num_pos * 10.0
    safe = jnp.where(denom == 0.0, jnp.float32(1.0), denom)
    l = loss_sum / safe
    ts = target_scores_sum.reshape(())
    lpos = jnp.where(ts == 0.0, l, l / ts)
    return jnp.where(num_pos > 0.0, lpos, jnp.float32(0.0))
